# baseline (speedup 1.0000x reference)
import functools

import jax
import jax.numpy as jnp
from jax import lax
from jax.experimental import pallas as pl
from jax.experimental.pallas import tpu as pltpu
from jax.experimental.pallas import tpu_sc as plsc

_NUM_CORES = 2
_NUM_SUBCORES = 16
_NW = _NUM_CORES * _NUM_SUBCORES

_CHUNK = 128
_NBUF = 2


@functools.lru_cache(maxsize=None)
def _build(B, V, D):
    assert B % _NW == 0
    b_per_w = B // _NW
    assert b_per_w % _CHUNK == 0
    n_chunks = b_per_w // _CHUNK
    assert n_chunks % _NBUF == 0

    mesh = plsc.VectorSubcoreMesh(
        core_axis_name="c",
        subcore_axis_name="s",
        num_cores=_NUM_CORES,
        num_subcores=_NUM_SUBCORES,
    )

    @functools.partial(
        pl.kernel,
        out_type=jax.ShapeDtypeStruct((B, D), jnp.float32),
        mesh=mesh,
        scratch_types=[
            pltpu.VMEM((b_per_w,), jnp.int32),
            pltpu.VMEM((_NBUF, _CHUNK, D), jnp.float32),
            pltpu.SemaphoreType.DMA,
            pltpu.SemaphoreType.DMA,
        ],
        compiler_params=pltpu.CompilerParams(use_tc_tiling_on_sc=False),
    )
    def emb_kernel(table_hbm, idx_hbm, out_hbm, idx_v, rows_v, sem0, sem1):
        sems = (sem0, sem1)
        wid = lax.axis_index("s") * _NUM_CORES + lax.axis_index("c")
        base = wid * b_per_w

        pltpu.sync_copy(idx_hbm.at[pl.ds(base, b_per_w)], idx_v)

        def start(g, b):
            pltpu.async_copy(
                table_hbm.at[idx_v.at[pl.ds(g * _CHUNK, _CHUNK)]],
                rows_v.at[b],
                sems[b],
            )

        def wait(b):
            pltpu.make_async_copy(
                table_hbm.at[pl.ds(0, _CHUNK)], rows_v.at[b], sems[b]
            ).wait()

        def writeout(g, b):
            pltpu.sync_copy(
                rows_v.at[b], out_hbm.at[pl.ds(base + g * _CHUNK, _CHUNK)]
            )

        start(0, 0)

        @pl.loop(0, n_chunks, step=_NBUF)
        def _(g):
            for b in range(_NBUF):
                nxt = g + b + 1

                @pl.when(nxt < n_chunks)
                def _():
                    start(nxt, (b + 1) % _NBUF)

                wait(b)
                writeout(g + b, b)

    return emb_kernel


def kernel(inputs, embeddings):
    V, D = embeddings.shape
    B = inputs.shape[0] * inputs.shape[1]
    idx = inputs.reshape(-1).astype(jnp.int32)
    out = _build(B, V, D)(embeddings, idx)
    return out.reshape(inputs.shape + (D,))


# speedup vs baseline: 1.0079x; 1.0079x over previous
import functools

import jax
import jax.numpy as jnp
from jax import lax
from jax.experimental import pallas as pl
from jax.experimental.pallas import tpu as pltpu
from jax.experimental.pallas import tpu_sc as plsc

_NUM_CORES = 2
_NUM_SUBCORES = 16
_NW = _NUM_CORES * _NUM_SUBCORES

_CHUNK = 128
_NBUF = 5


@functools.lru_cache(maxsize=None)
def _build(B, V, D):
    assert B % _NW == 0
    b_per_w = B // _NW
    assert b_per_w % _CHUNK == 0
    n_chunks = b_per_w // _CHUNK
    assert n_chunks % _NBUF == 0

    mesh = plsc.VectorSubcoreMesh(
        core_axis_name="c",
        subcore_axis_name="s",
        num_cores=_NUM_CORES,
        num_subcores=_NUM_SUBCORES,
    )

    @functools.partial(
        pl.kernel,
        out_type=jax.ShapeDtypeStruct((B, D), jnp.float32),
        mesh=mesh,
        scratch_types=[
            pltpu.VMEM((b_per_w,), jnp.int32),
            pltpu.VMEM((_NBUF, _CHUNK, D), jnp.float32),
        ]
        + [pltpu.SemaphoreType.DMA] * _NBUF,
        compiler_params=pltpu.CompilerParams(use_tc_tiling_on_sc=False),
    )
    def emb_kernel(table_hbm, idx_hbm, out_hbm, idx_v, rows_v, *sems):
        wid = lax.axis_index("s") * _NUM_CORES + lax.axis_index("c")
        base = wid * b_per_w

        pltpu.sync_copy(idx_hbm.at[pl.ds(base, b_per_w)], idx_v)

        def start(g, b):
            pltpu.async_copy(
                table_hbm.at[idx_v.at[pl.ds(g * _CHUNK, _CHUNK)]],
                rows_v.at[b],
                sems[b],
            )

        def wait(b):
            pltpu.make_async_copy(
                table_hbm.at[pl.ds(0, _CHUNK)], rows_v.at[b], sems[b]
            ).wait()

        def writeout(g, b):
            pltpu.sync_copy(
                rows_v.at[b], out_hbm.at[pl.ds(base + g * _CHUNK, _CHUNK)]
            )

        for b in range(_NBUF - 1):
            start(b, b)

        @pl.loop(0, n_chunks, step=_NBUF)
        def _(g):
            for b in range(_NBUF):
                nxt = g + b + _NBUF - 1

                @pl.when(nxt < n_chunks)
                def _():
                    start(nxt, (b + _NBUF - 1) % _NBUF)

                wait(b)
                writeout(g + b, b)

    return emb_kernel


def kernel(inputs, embeddings):
    V, D = embeddings.shape
    B = inputs.shape[0] * inputs.shape[1]
    idx = inputs.reshape(-1).astype(jnp.int32)
    out = _build(B, V, D)(embeddings, idx)
    return out.reshape(inputs.shape + (D,))
